# baseline (speedup 1.0000x reference)
import functools

import jax
import jax.numpy as jnp
from jax import lax
from jax.experimental import pallas as pl
from jax.experimental.pallas import tpu as pltpu
from jax.experimental.pallas import tpu_sc as plsc

NC = 2
NS = 16
NW = NC * NS

B = 16384
L = 20
T = 5
D = 64
W = 128
V = 1000000
NRB = V // W
VTAIL = NRB * W
PROWS = V // 2
LANES = 16
DV = D // LANES

RPW = B // NW
SB = 8
NSUB = RPW // SB
CI = SB * L
TI = SB * T

NBLK = (NRB + NW - 1) // NW

_SC_PARAMS = pltpu.CompilerParams(
    needs_layout_passes=False, use_tc_tiling_on_sc=True)
_MESH = dict(core_axis_name="c", subcore_axis_name="s")


def _repack_body(tt_hbm, app_hbm, p_hbm, inb, outb, si0, si1, so0, so1):
  wid = lax.axis_index("s") * NC + lax.axis_index("c")
  isems = (si0, si1)
  osems = (so0, so1)
  lanes = lax.iota(jnp.int32, LANES)
  perms = [(lanes + k) & (LANES - 1) for k in range(LANES)]
  drows = [lanes + bi * LANES for bi in range(D // LANES)]
  phalf = [p >> 1 for p in perms]
  pcol = [(p & 1) * D for p in perms]

  def blk(i):
    return wid + i * NW

  def fire_in(i, b):
    @pl.when(blk(i) < NRB)
    def _():
      for t8 in range(D // 8):
        pltpu.async_copy(
            tt_hbm.at[pl.ds(t8 * 8, 8), pl.ds(blk(i) * W, W)],
            inb.at[b].at[pl.ds(t8 * 8, 8)], isems[b])

  def drain_in(i, b):
    @pl.when(blk(i) < NRB)
    def _():
      for t8 in range(D // 8):
        pltpu.make_async_copy(
            tt_hbm.at[pl.ds(t8 * 8, 8), pl.ds(blk(i) * W, W)],
            inb.at[b].at[pl.ds(t8 * 8, 8)], isems[b]).wait()

  def fire_out(i, b):
    @pl.when(blk(i) < NRB)
    def _():
      pltpu.async_copy(
          outb.at[b], p_hbm.at[pl.ds(blk(i) * (W // 2), W // 2)], osems[b])

  def drain_out(i, b):
    @pl.when(blk(i) < NRB)
    def _():
      pltpu.make_async_copy(
          outb.at[b], p_hbm.at[pl.ds(blk(i) * (W // 2), W // 2)],
          osems[b]).wait()

  def transpose(b):
    src = inb.at[b]
    dst = outb.at[b]

    def tbody(rj, carry):
      r16 = rj * LANES
      cvs = [perms[k] + r16 for k in range(LANES)]
      rws = [phalf[k] + (rj * (LANES // 2)) for k in range(LANES)]
      for bi in range(D // LANES):
        colv = [pcol[k] + drows[bi] for k in range(LANES)]
        gs = [plsc.load_gather(src, [drows[bi], cvs[k]])
              for k in range(LANES)]
        for k in range(LANES):
          plsc.store_scatter(dst, [rws[k], colv[k]], gs[k])
      return carry

    lax.fori_loop(0, W // LANES, tbody, 0, unroll=1)

  @pl.when(wid == 0)
  def _():
    pltpu.sync_copy(app_hbm, p_hbm.at[pl.ds(VTAIL // 2, (V - VTAIL) // 2)])

  fire_in(0, 0)

  def outer(m, carry):
    i = m * 2
    fire_in(i + 1, 1)
    drain_in(i, 0)

    @pl.when(m > 0)
    def _():
      drain_out(i - 2, 0)
    transpose(0)
    fire_out(i, 0)

    fire_in(i + 2, 0)
    drain_in(i + 1, 1)

    @pl.when(m > 0)
    def _():
      drain_out(i - 1, 1)
    transpose(1)
    fire_out(i + 1, 1)
    return carry

  lax.fori_loop(0, NBLK // 2, outer, 0, unroll=1)
  last = NBLK - 1
  drain_in(last, 0)
  drain_out(last - 2, 0)
  drain_out(last - 1, 1)
  transpose(0)
  fire_out(last, 0)
  drain_out(last, 0)


def _cbow_body(ctx_hbm, cto_hbm, tgt_hbm, tto_hbm, emb_hbm, out_hbm,
               ctx_idx, ctx_off, tgt_idx, tgt_off,
               ctx_rows, tgt_rows, acc_t, out_tile, sem_g0, sem_g1):
  wid = lax.axis_index("s") * NC + lax.axis_index("c")
  gsems = (sem_g0, sem_g1)

  pltpu.sync_copy(ctx_hbm.at[pl.ds(wid * (RPW * L), RPW * L)], ctx_idx)
  pltpu.sync_copy(cto_hbm.at[pl.ds(wid * (RPW * L), RPW * L)],
                  ctx_off.at[pl.ds(0, RPW * L)])
  pltpu.sync_copy(tgt_hbm.at[pl.ds(wid * (RPW * T), RPW * T)], tgt_idx)
  pltpu.sync_copy(tto_hbm.at[pl.ds(wid * (RPW * T), RPW * T)],
                  tgt_off.at[pl.ds(0, RPW * T)])

  def gather_list(s, nb):
    c0 = s * CI
    t0 = s * TI
    return [
        (ctx_idx.at[pl.ds(c0, 128)], ctx_rows.at[nb].at[pl.ds(0, 128)]),
        (ctx_idx.at[pl.ds(c0 + 128, 32)], ctx_rows.at[nb].at[pl.ds(128, 32)]),
        (tgt_idx.at[pl.ds(t0, TI)], tgt_rows.at[nb]),
    ]

  def fire(s, nb):
    for idx, dst in gather_list(s, nb):
      pltpu.async_copy(emb_hbm.at[idx], dst, gsems[nb])

  def drain(s, buf):
    for idx, dst in gather_list(s, buf):
      pltpu.make_async_copy(emb_hbm.at[idx], dst, gsems[buf]).wait()

  def compute(s, buf):
    crows = ctx_rows.at[buf]
    trows = tgt_rows.at[buf]
    lanes = lax.iota(jnp.int32, LANES)
    c0 = s * CI
    t0 = s * TI

    def body(b, carry):
      cb = b * L
      offs1 = ctx_off[pl.ds(c0 + cb, LANES)]
      offs2 = ctx_off[pl.ds(c0 + cb + (L - LANES), LANES)]

      def coff(j):
        return offs1[j] if j < LANES else offs2[j - (L - LANES)]

      vc = [crows[cb, pl.ds(coff(0) + k * LANES, LANES)] for k in range(DV)]
      for j in range(1, L):
        oj = coff(j)
        for k in range(DV):
          vc[k] = vc[k] + crows[cb + j, pl.ds(oj + k * LANES, LANES)]
      scale = jnp.float32(1.0 / L)
      vc = [v * scale for v in vc]
      tb = b * T
      toffs = tgt_off[pl.ds(t0 + tb, LANES)]
      for t in range(T):
        ot = toffs[t]
        acc = vc[0] * trows[tb + t, pl.ds(ot, LANES)]
        for k in range(1, DV):
          acc = acc + vc[k] * trows[tb + t, pl.ds(ot + k * LANES, LANES)]
        pair = jnp.full((LANES,), tb + t, dtype=jnp.int32)
        plsc.store_scatter(acc_t, [lanes, pair], acc)
      return carry

    lax.fori_loop(0, SB, body, 0, unroll=1)

    for g in range((TI + LANES - 1) // LANES):
      p0 = g * LANES
      tot = acc_t[0, pl.ds(p0, LANES)]
      for l in range(1, LANES):
        tot = tot + acc_t[l, pl.ds(p0, LANES)]
      out_tile[pl.ds(p0, LANES)] = tot

    e0 = (wid * RPW + s * SB) * T
    pltpu.sync_copy(out_tile.at[pl.ds(0, TI)], out_hbm.at[pl.ds(e0, TI)])

  fire(0, 0)

  def outer(m, carry):
    s = m * 2
    fire(s + 1, 1)
    drain(s, 0)
    compute(s, 0)
    fire(s + 2, 0)
    drain(s + 1, 1)
    compute(s + 1, 1)
    return carry

  lax.fori_loop(0, NSUB // 2 - 1, outer, 0, unroll=1)
  s = NSUB - 2
  fire(s + 1, 1)
  drain(s, 0)
  compute(s, 0)
  drain(s + 1, 1)
  compute(s + 1, 1)


@jax.jit
def kernel(context, targets, embedding):
  ctx = context.astype(jnp.int32)
  tgt = targets.astype(jnp.int32)
  ctx_phys = (ctx >> 1).reshape(-1)
  ctx_off = ((ctx & 1) * D).reshape(-1)
  tgt_phys = (tgt >> 1).reshape(-1)
  tgt_off = ((tgt & 1) * D).reshape(-1)
  tt = embedding.T
  appendix = embedding[VTAIL:].reshape(-1, W)

  repack = functools.partial(
      pl.kernel,
      out_type=jax.ShapeDtypeStruct((PROWS, W), jnp.float32),
      mesh=plsc.VectorSubcoreMesh(**_MESH),
      compiler_params=_SC_PARAMS,
      scratch_types=[
          pltpu.VMEM((2, D, W), jnp.float32),
          pltpu.VMEM((2, W // 2, W), jnp.float32),
          pltpu.SemaphoreType.DMA,
          pltpu.SemaphoreType.DMA,
          pltpu.SemaphoreType.DMA,
          pltpu.SemaphoreType.DMA,
      ],
  )(_repack_body)
  table = repack(tt, appendix)

  score = functools.partial(
      pl.kernel,
      out_type=jax.ShapeDtypeStruct((B * T,), jnp.float32),
      mesh=plsc.VectorSubcoreMesh(**_MESH),
      compiler_params=_SC_PARAMS,
      scratch_types=[
          pltpu.VMEM((RPW * L,), jnp.int32),
          pltpu.VMEM((RPW * L + LANES,), jnp.int32),
          pltpu.VMEM((RPW * T,), jnp.int32),
          pltpu.VMEM((RPW * T + LANES,), jnp.int32),
          pltpu.VMEM((2, CI, W), jnp.float32),
          pltpu.VMEM((2, TI, W), jnp.float32),
          pltpu.VMEM((LANES, 48), jnp.float32),
          pltpu.VMEM((48,), jnp.float32),
          pltpu.SemaphoreType.DMA,
          pltpu.SemaphoreType.DMA,
      ],
  )(_cbow_body)
  return score(ctx_phys, ctx_off, tgt_phys, tgt_off, table).reshape(B, T)


# speedup vs baseline: 1.0388x; 1.0388x over previous
import functools

import jax
import jax.numpy as jnp
from jax import lax
from jax.experimental import pallas as pl
from jax.experimental.pallas import tpu as pltpu
from jax.experimental.pallas import tpu_sc as plsc

NC = 2
NS = 16
NW = NC * NS

B = 16384
L = 20
T = 5
D = 64
W = 128
V = 1000000
NRB = V // W
VTAIL = NRB * W
VPAD = VTAIL + W
LANES = 16
DV = D // LANES

RPW = B // NW
SB = 16
NSUB = RPW // SB
CI = SB * L
TI = SB * T

NBLK = (NRB + NW - 1) // NW

_SC_PARAMS = pltpu.CompilerParams(
    needs_layout_passes=False, use_tc_tiling_on_sc=True)
_MESH = dict(core_axis_name="c", subcore_axis_name="s")


def _repack_body(tt_hbm, app_hbm, p_hbm, inb, outb, si0, si1, so0, so1):
  wid = lax.axis_index("s") * NC + lax.axis_index("c")
  isems = (si0, si1)
  osems = (so0, so1)
  lanes = lax.iota(jnp.int32, LANES)
  perms = [(lanes + k) & (LANES - 1) for k in range(LANES)]
  drows = [lanes + bi * LANES for bi in range(D // LANES)]

  def blk(i):
    return wid + i * NW

  def fire_in(i, b):
    @pl.when(blk(i) < NRB)
    def _():
      for t8 in range(D // 8):
        pltpu.async_copy(
            tt_hbm.at[pl.ds(t8 * 8, 8), pl.ds(blk(i) * W, W)],
            inb.at[b].at[pl.ds(t8 * 8, 8)], isems[b])

  def drain_in(i, b):
    @pl.when(blk(i) < NRB)
    def _():
      for t8 in range(D // 8):
        pltpu.make_async_copy(
            tt_hbm.at[pl.ds(t8 * 8, 8), pl.ds(blk(i) * W, W)],
            inb.at[b].at[pl.ds(t8 * 8, 8)], isems[b]).wait()

  def fire_out(i, b):
    @pl.when(blk(i) < NRB)
    def _():
      pltpu.async_copy(
          outb.at[b], p_hbm.at[pl.ds(blk(i) * W, W)], osems[b])

  def drain_out(i, b):
    @pl.when(blk(i) < NRB)
    def _():
      pltpu.make_async_copy(
          outb.at[b], p_hbm.at[pl.ds(blk(i) * W, W)], osems[b]).wait()

  def transpose(b):
    src = inb.at[b]
    dst = outb.at[b]

    def tbody(rj, carry):
      r16 = rj * LANES
      cvs = [perms[k] + r16 for k in range(LANES)]
      for bh in range(D // LANES // 2):
        bis = (2 * bh, 2 * bh + 1)
        gs = [plsc.load_gather(src, [drows[bi], cvs[k]])
              for bi in bis for k in range(LANES)]
        for n, bi in enumerate(bis):
          for k in range(LANES):
            plsc.store_scatter(dst, [cvs[k], drows[bi]],
                               gs[n * LANES + k])
      return carry

    lax.fori_loop(0, W // LANES, tbody, 0, unroll=1)

  @pl.when(wid == 0)
  def _():
    pltpu.sync_copy(app_hbm, p_hbm.at[pl.ds(VTAIL, V - VTAIL)])

  fire_in(0, 0)

  def outer(m, carry):
    i = m * 2
    fire_in(i + 1, 1)
    drain_in(i, 0)

    @pl.when(m > 0)
    def _():
      drain_out(i - 2, 0)
    transpose(0)
    fire_out(i, 0)

    fire_in(i + 2, 0)
    drain_in(i + 1, 1)

    @pl.when(m > 0)
    def _():
      drain_out(i - 1, 1)
    transpose(1)
    fire_out(i + 1, 1)
    return carry

  lax.fori_loop(0, NBLK // 2, outer, 0, unroll=1)
  last = NBLK - 1
  drain_in(last, 0)
  drain_out(last - 2, 0)
  drain_out(last - 1, 1)
  transpose(0)
  fire_out(last, 0)
  drain_out(last, 0)


def _cbow_body(ctx_hbm, tgt_hbm, emb_hbm, out_hbm,
               ctx_idx, tgt_idx, ctx_rows, tgt_rows, acc_t, out_tile,
               sem_g0, sem_g1):
  wid = lax.axis_index("s") * NC + lax.axis_index("c")
  gsems = (sem_g0, sem_g1)

  pltpu.sync_copy(ctx_hbm.at[pl.ds(wid * (RPW * L), RPW * L)], ctx_idx)
  pltpu.sync_copy(tgt_hbm.at[pl.ds(wid * (RPW * T), RPW * T)], tgt_idx)

  def gather_list(s, nb):
    c0 = s * CI
    t0 = s * TI
    return [
        (ctx_idx.at[pl.ds(c0, 128)], ctx_rows.at[nb].at[pl.ds(0, 128)]),
        (ctx_idx.at[pl.ds(c0 + 128, 128)], ctx_rows.at[nb].at[pl.ds(128, 128)]),
        (ctx_idx.at[pl.ds(c0 + 256, 64)], ctx_rows.at[nb].at[pl.ds(256, 64)]),
        (tgt_idx.at[pl.ds(t0, TI)], tgt_rows.at[nb]),
    ]

  def fire(s, nb):
    for idx, dst in gather_list(s, nb):
      pltpu.async_copy(emb_hbm.at[idx], dst, gsems[nb])

  def drain(s, buf):
    for idx, dst in gather_list(s, buf):
      pltpu.make_async_copy(emb_hbm.at[idx], dst, gsems[buf]).wait()

  def compute(s, buf):
    crows = ctx_rows.at[buf]
    trows = tgt_rows.at[buf]
    lanes = lax.iota(jnp.int32, LANES)

    def body(b, carry):
      cb = b * L
      vc = [crows[cb, pl.ds(k * LANES, LANES)] for k in range(DV)]
      for j in range(1, L):
        for k in range(DV):
          vc[k] = vc[k] + crows[cb + j, pl.ds(k * LANES, LANES)]
      scale = jnp.float32(1.0 / L)
      vc = [v * scale for v in vc]
      tb = b * T
      for t in range(T):
        acc = vc[0] * trows[tb + t, pl.ds(0, LANES)]
        for k in range(1, DV):
          acc = acc + vc[k] * trows[tb + t, pl.ds(k * LANES, LANES)]
        pair = jnp.full((LANES,), tb + t, dtype=jnp.int32)
        plsc.store_scatter(acc_t, [lanes, pair], acc)
      return carry

    lax.fori_loop(0, SB, body, 0, unroll=1)

    for g in range(TI // LANES):
      p0 = g * LANES
      tot = acc_t[0, pl.ds(p0, LANES)]
      for l in range(1, LANES):
        tot = tot + acc_t[l, pl.ds(p0, LANES)]
      out_tile[pl.ds(p0, LANES)] = tot

    e0 = (wid * RPW + s * SB) * T
    pltpu.sync_copy(out_tile, out_hbm.at[pl.ds(e0, TI)])

  fire(0, 0)

  def outer(m, carry):
    s = m * 2
    fire(s + 1, 1)
    drain(s, 0)
    compute(s, 0)
    fire(s + 2, 0)
    drain(s + 1, 1)
    compute(s + 1, 1)
    return carry

  lax.fori_loop(0, NSUB // 2 - 1, outer, 0, unroll=1)
  s = NSUB - 2
  fire(s + 1, 1)
  drain(s, 0)
  compute(s, 0)
  drain(s + 1, 1)
  compute(s + 1, 1)


@jax.jit
def kernel(context, targets, embedding):
  ctx_flat = context.astype(jnp.int32).reshape(-1)
  tgt_flat = targets.astype(jnp.int32).reshape(-1)
  tt = embedding.T
  appendix = jnp.pad(embedding[VTAIL:], ((0, 0), (0, W - D)))

  repack = functools.partial(
      pl.kernel,
      out_type=jax.ShapeDtypeStruct((VPAD, W), jnp.float32),
      mesh=plsc.VectorSubcoreMesh(**_MESH),
      compiler_params=_SC_PARAMS,
      scratch_types=[
          pltpu.VMEM((2, D, W), jnp.float32),
          pltpu.VMEM((2, W, W), jnp.float32),
          pltpu.SemaphoreType.DMA,
          pltpu.SemaphoreType.DMA,
          pltpu.SemaphoreType.DMA,
          pltpu.SemaphoreType.DMA,
      ],
  )(_repack_body)
  table = repack(tt, appendix)

  score = functools.partial(
      pl.kernel,
      out_type=jax.ShapeDtypeStruct((B * T,), jnp.float32),
      mesh=plsc.VectorSubcoreMesh(**_MESH),
      compiler_params=_SC_PARAMS,
      scratch_types=[
          pltpu.VMEM((RPW * L,), jnp.int32),
          pltpu.VMEM((RPW * T,), jnp.int32),
          pltpu.VMEM((2, CI, W), jnp.float32),
          pltpu.VMEM((2, TI, W), jnp.float32),
          pltpu.VMEM((LANES, TI), jnp.float32),
          pltpu.VMEM((TI,), jnp.float32),
          pltpu.SemaphoreType.DMA,
          pltpu.SemaphoreType.DMA,
      ],
  )(_cbow_body)
  return score(ctx_flat, tgt_flat, table).reshape(B, T)


# speedup vs baseline: 1.1176x; 1.0759x over previous
import functools

import jax
import jax.numpy as jnp
from jax import lax
from jax.experimental import pallas as pl
from jax.experimental.pallas import tpu as pltpu
from jax.experimental.pallas import tpu_sc as plsc

NC = 2
NS = 16
NW = NC * NS

B = 16384
L = 20
T = 5
D = 64
W = 128
V = 1000000
NRB = V // W
VTAIL = NRB * W
VPAD = VTAIL + W
LANES = 16
DV = D // LANES

RPW = B // NW
SB = 16
NSUB = RPW // SB
CI = SB * L
TI = SB * T

NBLK = (NRB + NW - 1) // NW

_SC_PARAMS = pltpu.CompilerParams(
    needs_layout_passes=False, use_tc_tiling_on_sc=True)
_MESH = dict(core_axis_name="c", subcore_axis_name="s")


def _repack_body(tt_hbm, app_hbm, p_hbm, inb, outb, si0, si1, so0, so1):
  wid = lax.axis_index("s") * NC + lax.axis_index("c")
  isems = (si0, si1)
  osems = (so0, so1)
  lanes = lax.iota(jnp.int32, LANES)
  perms = [(lanes + k) & (LANES - 1) for k in range(LANES)]
  drows = [lanes + bi * LANES for bi in range(D // LANES)]

  def blk(i):
    return wid + i * NW

  def fire_in(i, b):
    @pl.when(blk(i) < NRB)
    def _():
      for t8 in range(D // 8):
        pltpu.async_copy(
            tt_hbm.at[pl.ds(t8 * 8, 8), pl.ds(blk(i) * W, W)],
            inb.at[b].at[pl.ds(t8 * 8, 8)], isems[b])

  def drain_in(i, b):
    @pl.when(blk(i) < NRB)
    def _():
      for t8 in range(D // 8):
        pltpu.make_async_copy(
            tt_hbm.at[pl.ds(t8 * 8, 8), pl.ds(blk(i) * W, W)],
            inb.at[b].at[pl.ds(t8 * 8, 8)], isems[b]).wait()

  def fire_out(i, b):
    @pl.when(blk(i) < NRB)
    def _():
      pltpu.async_copy(
          outb.at[b], p_hbm.at[pl.ds(blk(i) * W, W)], osems[b])

  def drain_out(i, b):
    @pl.when(blk(i) < NRB)
    def _():
      pltpu.make_async_copy(
          outb.at[b], p_hbm.at[pl.ds(blk(i) * W, W)], osems[b]).wait()

  def transpose(b):
    src = inb.at[b]
    dst = outb.at[b]

    def tbody(rj, carry):
      r16 = rj * LANES
      cvs = [perms[k] + r16 for k in range(LANES)]
      for bi in range(D // LANES):
        gs = [plsc.load_gather(src, [drows[bi], cvs[k]])
              for k in range(LANES)]
        for k in range(LANES):
          plsc.store_scatter(dst, [cvs[k], drows[bi]], gs[k])
      return carry

    lax.fori_loop(0, W // LANES, tbody, 0, unroll=1)

  @pl.when(wid == 0)
  def _():
    pltpu.sync_copy(app_hbm, p_hbm.at[pl.ds(VTAIL, V - VTAIL)])

  fire_in(0, 0)

  def outer(m, carry):
    i = m * 2
    fire_in(i + 1, 1)
    drain_in(i, 0)

    @pl.when(m > 0)
    def _():
      drain_out(i - 2, 0)
    transpose(0)
    fire_out(i, 0)

    fire_in(i + 2, 0)
    drain_in(i + 1, 1)

    @pl.when(m > 0)
    def _():
      drain_out(i - 1, 1)
    transpose(1)
    fire_out(i + 1, 1)
    return carry

  lax.fori_loop(0, NBLK // 2, outer, 0, unroll=1)
  last = NBLK - 1
  drain_in(last, 0)
  drain_out(last - 2, 0)
  drain_out(last - 1, 1)
  transpose(0)
  fire_out(last, 0)
  drain_out(last, 0)


def _cbow_body(ctx_hbm, tgt_hbm, emb_hbm, out_hbm,
               ctx_idx, tgt_idx, ctx_rows, tgt_rows, acc_t, out_tile,
               sem_g0, sem_g1):
  wid = lax.axis_index("s") * NC + lax.axis_index("c")
  gsems = (sem_g0, sem_g1)

  pltpu.sync_copy(ctx_hbm.at[pl.ds(wid * (RPW * L), RPW * L)], ctx_idx)
  pltpu.sync_copy(tgt_hbm.at[pl.ds(wid * (RPW * T), RPW * T)], tgt_idx)

  def gather_list(s, nb):
    c0 = s * CI
    t0 = s * TI
    return [
        (ctx_idx.at[pl.ds(c0, 128)], ctx_rows.at[nb].at[pl.ds(0, 128)]),
        (ctx_idx.at[pl.ds(c0 + 128, 128)], ctx_rows.at[nb].at[pl.ds(128, 128)]),
        (ctx_idx.at[pl.ds(c0 + 256, 64)], ctx_rows.at[nb].at[pl.ds(256, 64)]),
        (tgt_idx.at[pl.ds(t0, TI)], tgt_rows.at[nb]),
    ]

  def fire(s, nb):
    for idx, dst in gather_list(s, nb):
      pltpu.async_copy(emb_hbm.at[idx], dst, gsems[nb])

  def drain(s, buf):
    for idx, dst in gather_list(s, buf):
      pltpu.make_async_copy(emb_hbm.at[idx], dst, gsems[buf]).wait()

  def compute(s, buf):
    crows = ctx_rows.at[buf]
    trows = tgt_rows.at[buf]
    lanes = lax.iota(jnp.int32, LANES)

    def body(b, carry):
      cb = b * L
      vc = [crows[cb, pl.ds(k * LANES, LANES)] for k in range(DV)]
      for j in range(1, L):
        for k in range(DV):
          vc[k] = vc[k] + crows[cb + j, pl.ds(k * LANES, LANES)]
      scale = jnp.float32(1.0 / L)
      vc = [v * scale for v in vc]
      tb = b * T
      for t in range(T):
        acc = vc[0] * trows[tb + t, pl.ds(0, LANES)]
        for k in range(1, DV):
          acc = acc + vc[k] * trows[tb + t, pl.ds(k * LANES, LANES)]
        pair = jnp.full((LANES,), tb + t, dtype=jnp.int32)
        plsc.store_scatter(acc_t, [lanes, pair], acc)
      return carry

    lax.fori_loop(0, SB, body, 0, unroll=1)

    for g in range(TI // LANES):
      p0 = g * LANES
      tot = acc_t[0, pl.ds(p0, LANES)]
      for l in range(1, LANES):
        tot = tot + acc_t[l, pl.ds(p0, LANES)]
      out_tile[pl.ds(p0, LANES)] = tot

    e0 = (wid * RPW + s * SB) * T
    pltpu.sync_copy(out_tile, out_hbm.at[pl.ds(e0, TI)])

  fire(0, 0)

  def outer(m, carry):
    s = m * 2
    fire(s + 1, 1)
    drain(s, 0)
    compute(s, 0)
    fire(s + 2, 0)
    drain(s + 1, 1)
    compute(s + 1, 1)
    return carry

  lax.fori_loop(0, NSUB // 2 - 1, outer, 0, unroll=1)
  s = NSUB - 2
  fire(s + 1, 1)
  drain(s, 0)
  compute(s, 0)
  drain(s + 1, 1)
  compute(s + 1, 1)


@jax.jit
def kernel(context, targets, embedding):
  ctx_flat = context.astype(jnp.int32).reshape(-1)
  tgt_flat = targets.astype(jnp.int32).reshape(-1)
  tt = embedding.T
  appendix = jnp.pad(embedding[VTAIL:], ((0, 0), (0, W - D)))

  repack = functools.partial(
      pl.kernel,
      out_type=jax.ShapeDtypeStruct((VPAD, W), jnp.float32),
      mesh=plsc.VectorSubcoreMesh(**_MESH),
      compiler_params=_SC_PARAMS,
      scratch_types=[
          pltpu.VMEM((2, D, W), jnp.float32),
          pltpu.VMEM((2, W, W), jnp.float32),
          pltpu.SemaphoreType.DMA,
          pltpu.SemaphoreType.DMA,
          pltpu.SemaphoreType.DMA,
          pltpu.SemaphoreType.DMA,
      ],
  )(_repack_body)
  table = repack(tt, appendix)

  score = functools.partial(
      pl.kernel,
      out_type=jax.ShapeDtypeStruct((B * T,), jnp.float32),
      mesh=plsc.VectorSubcoreMesh(**_MESH),
      compiler_params=_SC_PARAMS,
      scratch_types=[
          pltpu.VMEM((RPW * L,), jnp.int32),
          pltpu.VMEM((RPW * T,), jnp.int32),
          pltpu.VMEM((2, CI, W), jnp.float32),
          pltpu.VMEM((2, TI, W), jnp.float32),
          pltpu.VMEM((LANES, TI), jnp.float32),
          pltpu.VMEM((TI,), jnp.float32),
          pltpu.SemaphoreType.DMA,
          pltpu.SemaphoreType.DMA,
      ],
  )(_cbow_body)
  return score(ctx_flat, tgt_flat, table).reshape(B, T)


# speedup vs baseline: 1.1269x; 1.0083x over previous
import functools

import jax
import jax.numpy as jnp
from jax import lax
from jax.experimental import pallas as pl
from jax.experimental.pallas import tpu as pltpu
from jax.experimental.pallas import tpu_sc as plsc

NC = 2
NS = 16
NW = NC * NS

B = 16384
L = 20
T = 5
D = 64
W = 128
V = 1000000
NRB = V // W
VTAIL = NRB * W
VPAD = VTAIL + W
LANES = 16
DV = D // LANES

RPW = B // NW
SB = 16
NSUB = RPW // SB
CI = SB * L
TI = SB * T

NBLK = (NRB + NW - 1) // NW

_SC_PARAMS = pltpu.CompilerParams(
    needs_layout_passes=False, use_tc_tiling_on_sc=True)
_MESH = dict(core_axis_name="c", subcore_axis_name="s")


def _lane_permute(x, idx):
  return lax.gather(
      x, idx[:, None],
      dimension_numbers=lax.GatherDimensionNumbers(
          offset_dims=(), collapsed_slice_dims=(0,), start_index_map=(0,)),
      slice_sizes=(1,),
      mode=lax.GatherScatterMode.PROMISE_IN_BOUNDS)


def _repack_body(tt_hbm, app_hbm, p_hbm, inb, outb, si0, si1, so0, so1):
  wid = lax.axis_index("s") * NC + lax.axis_index("c")
  isems = (si0, si1)
  osems = (so0, so1)
  lanes = lax.iota(jnp.int32, LANES)
  perms = [(lanes + k) & (LANES - 1) for k in range(LANES)]
  drows = [lanes + bi * LANES for bi in range(D // LANES)]

  def blk(i):
    return wid + i * NW

  def fire_in(i, b):
    @pl.when(blk(i) < NRB)
    def _():
      for t8 in range(D // 8):
        pltpu.async_copy(
            tt_hbm.at[pl.ds(t8 * 8, 8), pl.ds(blk(i) * W, W)],
            inb.at[b].at[pl.ds(t8 * 8, 8)], isems[b])

  def drain_in(i, b):
    @pl.when(blk(i) < NRB)
    def _():
      for t8 in range(D // 8):
        pltpu.make_async_copy(
            tt_hbm.at[pl.ds(t8 * 8, 8), pl.ds(blk(i) * W, W)],
            inb.at[b].at[pl.ds(t8 * 8, 8)], isems[b]).wait()

  def fire_out(i, b):
    @pl.when(blk(i) < NRB)
    def _():
      pltpu.async_copy(
          outb.at[b], p_hbm.at[pl.ds(blk(i) * W, W)], osems[b])

  def drain_out(i, b):
    @pl.when(blk(i) < NRB)
    def _():
      pltpu.make_async_copy(
          outb.at[b], p_hbm.at[pl.ds(blk(i) * W, W)], osems[b]).wait()

  def transpose(b):
    src = inb.at[b]
    dst = outb.at[b]

    def tbody(rj, carry):
      r16 = rj * LANES
      cvs = [perms[k] + r16 for k in range(LANES)]
      for bi in range(D // LANES):
        gs = [plsc.load_gather(src, [drows[bi], cvs[k]])
              for k in range(LANES)]
        for k in range(LANES):
          plsc.store_scatter(dst, [cvs[k], drows[bi]], gs[k])
      return carry

    lax.fori_loop(0, W // LANES, tbody, 0, unroll=1)

  @pl.when(wid == 0)
  def _():
    pltpu.sync_copy(app_hbm, p_hbm.at[pl.ds(VTAIL, V - VTAIL)])

  fire_in(0, 0)

  def outer(m, carry):
    i = m * 2
    fire_in(i + 1, 1)
    drain_in(i, 0)

    @pl.when(m > 0)
    def _():
      drain_out(i - 2, 0)
    transpose(0)
    fire_out(i, 0)

    fire_in(i + 2, 0)
    drain_in(i + 1, 1)

    @pl.when(m > 0)
    def _():
      drain_out(i - 1, 1)
    transpose(1)
    fire_out(i + 1, 1)
    return carry

  lax.fori_loop(0, NBLK // 2, outer, 0, unroll=1)
  last = NBLK - 1
  drain_in(last, 0)
  drain_out(last - 2, 0)
  drain_out(last - 1, 1)
  transpose(0)
  fire_out(last, 0)
  drain_out(last, 0)


def _cbow_body(ctx_hbm, tgt_hbm, emb_hbm, out_hbm,
               ctx_idx, tgt_idx, ctx_rows, tgt_rows, acc_t, out_tile,
               sem_g0, sem_g1):
  wid = lax.axis_index("s") * NC + lax.axis_index("c")
  gsems = (sem_g0, sem_g1)

  pltpu.sync_copy(ctx_hbm.at[pl.ds(wid * (RPW * L), RPW * L)], ctx_idx)
  pltpu.sync_copy(tgt_hbm.at[pl.ds(wid * (RPW * T), RPW * T)], tgt_idx)

  def gather_list(s, nb):
    c0 = s * CI
    t0 = s * TI
    return [
        (ctx_idx.at[pl.ds(c0, 128)], ctx_rows.at[nb].at[pl.ds(0, 128)]),
        (ctx_idx.at[pl.ds(c0 + 128, 128)], ctx_rows.at[nb].at[pl.ds(128, 128)]),
        (ctx_idx.at[pl.ds(c0 + 256, 64)], ctx_rows.at[nb].at[pl.ds(256, 64)]),
        (tgt_idx.at[pl.ds(t0, TI)], tgt_rows.at[nb]),
    ]

  def fire(s, nb):
    for idx, dst in gather_list(s, nb):
      pltpu.async_copy(emb_hbm.at[idx], dst, gsems[nb])

  def drain(s, buf):
    for idx, dst in gather_list(s, buf):
      pltpu.make_async_copy(emb_hbm.at[idx], dst, gsems[buf]).wait()

  def compute(s, buf):
    crows = ctx_rows.at[buf]
    trows = tgt_rows.at[buf]
    lanes = lax.iota(jnp.int32, LANES)
    perms = [(lanes + k) & (LANES - 1) for k in range(LANES)]

    def body(b, carry):
      cb = b * L
      vc = [crows[cb, pl.ds(k * LANES, LANES)] for k in range(DV)]
      for j in range(1, L):
        for k in range(DV):
          vc[k] = vc[k] + crows[cb + j, pl.ds(k * LANES, LANES)]
      scale = jnp.float32(1.0 / L)
      vc = [v * scale for v in vc]
      tb = b * T
      for t in range(T):
        acc = vc[0] * trows[tb + t, pl.ds(0, LANES)]
        for k in range(1, DV):
          acc = acc + vc[k] * trows[tb + t, pl.ds(k * LANES, LANES)]
        pv = jnp.full((LANES,), tb + t, dtype=jnp.int32)
        col = (pv & ~(LANES - 1)) + ((pv + lanes) & (LANES - 1))
        plsc.store_scatter(acc_t, [lanes, col], acc)
      return carry

    lax.fori_loop(0, SB, body, 0, unroll=1)

    for g in range(TI // LANES):
      p0 = g * LANES
      tot = acc_t[0, pl.ds(p0, LANES)]
      for l in range(1, LANES):
        tot = tot + _lane_permute(acc_t[l, pl.ds(p0, LANES)], perms[l])
      out_tile[pl.ds(p0, LANES)] = tot

    e0 = (wid * RPW + s * SB) * T
    pltpu.sync_copy(out_tile, out_hbm.at[pl.ds(e0, TI)])

  fire(0, 0)

  def outer(m, carry):
    s = m * 2
    fire(s + 1, 1)
    drain(s, 0)
    compute(s, 0)
    fire(s + 2, 0)
    drain(s + 1, 1)
    compute(s + 1, 1)
    return carry

  lax.fori_loop(0, NSUB // 2 - 1, outer, 0, unroll=1)
  s = NSUB - 2
  fire(s + 1, 1)
  drain(s, 0)
  compute(s, 0)
  drain(s + 1, 1)
  compute(s + 1, 1)


@jax.jit
def kernel(context, targets, embedding):
  ctx_flat = context.astype(jnp.int32).reshape(-1)
  tgt_flat = targets.astype(jnp.int32).reshape(-1)
  tt = embedding.T
  appendix = jnp.pad(embedding[VTAIL:], ((0, 0), (0, W - D)))

  repack = functools.partial(
      pl.kernel,
      out_type=jax.ShapeDtypeStruct((VPAD, W), jnp.float32),
      mesh=plsc.VectorSubcoreMesh(**_MESH),
      compiler_params=_SC_PARAMS,
      scratch_types=[
          pltpu.VMEM((2, D, W), jnp.float32),
          pltpu.VMEM((2, W, W), jnp.float32),
          pltpu.SemaphoreType.DMA,
          pltpu.SemaphoreType.DMA,
          pltpu.SemaphoreType.DMA,
          pltpu.SemaphoreType.DMA,
      ],
  )(_repack_body)
  table = repack(tt, appendix)

  score = functools.partial(
      pl.kernel,
      out_type=jax.ShapeDtypeStruct((B * T,), jnp.float32),
      mesh=plsc.VectorSubcoreMesh(**_MESH),
      compiler_params=_SC_PARAMS,
      scratch_types=[
          pltpu.VMEM((RPW * L,), jnp.int32),
          pltpu.VMEM((RPW * T,), jnp.int32),
          pltpu.VMEM((2, CI, W), jnp.float32),
          pltpu.VMEM((2, TI, W), jnp.float32),
          pltpu.VMEM((LANES, TI), jnp.float32),
          pltpu.VMEM((TI,), jnp.float32),
          pltpu.SemaphoreType.DMA,
          pltpu.SemaphoreType.DMA,
      ],
  )(_cbow_body)
  return score(ctx_flat, tgt_flat, table).reshape(B, T)
